# baseline (speedup 1.0000x reference)
import functools

import numpy as np
import jax
import jax.numpy as jnp
from jax import lax
from jax.experimental import pallas as pl
from jax.experimental.pallas import tpu as pltpu
from jax.experimental.pallas import tpu_sc as plsc

JOINTS = 22
EMB = 2048
NC = 4
SPLIT = 2
NDMA = NC * SPLIT
RQ = EMB // NC
RD = RQ // SPLIT

NUM_CORES = 2
NUM_SUBCORES = 16
NW = NUM_CORES * NUM_SUBCORES
BP = 512
BPW = BP // NW
L = 16

_idx = np.zeros((NW, 2 * BPW), dtype=np.int32)
for _w in range(NW):
    for _k in range(BPW):
        _p = _w * BPW + _k
        _i, _j = divmod(_p, JOINTS) if _p < JOINTS * JOINTS else (0, 0)
        _idx[_w, _k] = _j
        _idx[_w, BPW + _k] = JOINTS + _i
IDX_TABLE = jnp.asarray(_idx)


def _matmul_kernel(src_ref, b_ref, w_hbm, yy_ref, w_vmem, in_sems):
    copies = []
    for d in range(NDMA):
        cp = pltpu.make_async_copy(
            w_hbm.at[pl.ds(d * RD, RD), :],
            w_vmem.at[d // SPLIT, pl.ds((d % SPLIT) * RD, RD), :],
            in_sems.at[d])
        cp.start()
        copies.append(cp)
    src = src_ref[0]
    for q in range(NC):
        for s in range(SPLIT):
            copies[q * SPLIT + s].wait()
        y = jax.lax.dot_general(
            src, w_vmem[q],
            dimension_numbers=(((1,), (1,)), ((), ())),
            preferred_element_type=jnp.float32,
        )
        yy_ref[:JOINTS, q * RQ:(q + 1) * RQ] = y + b_ref[:, q * RQ:(q + 1) * RQ]
        yy_ref[JOINTS:, q * RQ:(q + 1) * RQ] = y


_sc_mesh = plsc.VectorSubcoreMesh(core_axis_name="c", subcore_axis_name="s")


@functools.partial(
    pl.kernel, mesh=_sc_mesh,
    out_type=jax.ShapeDtypeStruct((BP, EMB), jnp.float32),
    scratch_types=[
        pltpu.VMEM((2 * BPW,), jnp.int32),
        pltpu.VMEM((2 * BPW, EMB), jnp.float32),
        pltpu.SemaphoreType.DMA,
    ],
)
def _sc_expand(yy_hbm, idx_hbm, out_hbm, idx_v, rows_v, sem):
    wid = lax.axis_index("s") * NUM_CORES + lax.axis_index("c")
    base = wid * BPW
    pltpu.sync_copy(idx_hbm.at[wid], idx_v)
    pltpu.async_copy(yy_hbm.at[idx_v], rows_v, sem).wait()

    def body(t, carry):
        r = t // (EMB // L)
        c = (t % (EMB // L)) * L
        rows_v[r, pl.ds(c, L)] = rows_v[r, pl.ds(c, L)] - rows_v[r + BPW, pl.ds(c, L)]
        return carry

    lax.fori_loop(0, BPW * (EMB // L), body, 0)
    pltpu.sync_copy(rows_v.at[pl.ds(0, BPW)], out_hbm.at[pl.ds(base, BPW)])


def kernel(src, W, b):
    yy = pl.pallas_call(
        _matmul_kernel,
        in_specs=[
            pl.BlockSpec((1, JOINTS, EMB), lambda: (0, 0, 0)),
            pl.BlockSpec((1, EMB), lambda: (0, 0)),
            pl.BlockSpec(memory_space=pltpu.MemorySpace.HBM),
        ],
        out_specs=pl.BlockSpec((2 * JOINTS, EMB), lambda: (0, 0)),
        out_shape=jax.ShapeDtypeStruct((2 * JOINTS, EMB), jnp.float32),
        scratch_shapes=[
            pltpu.VMEM((NC, RQ, EMB), jnp.float32),
            pltpu.SemaphoreType.DMA((NDMA,)),
        ],
    )(src, b.reshape(1, EMB), W)
    out2 = _sc_expand(yy, IDX_TABLE)
    return out2[:JOINTS * JOINTS].reshape(JOINTS, JOINTS, EMB)


# speedup vs baseline: 4.6767x; 4.6767x over previous
import jax
import jax.numpy as jnp
from jax.experimental import pallas as pl
from jax.experimental.pallas import tpu as pltpu

JOINTS = 22
EMB = 2048
NC = 4
SPLIT = 2
NDMA = NC * SPLIT
RQ = EMB // NC
RD = RQ // SPLIT


def _edge_enhance_kernel(src_ref, b_ref, w_hbm, out_hbm, w_vmem, out_vmem,
                         in_sems, out_sems):
    copies = []
    for d in range(NDMA):
        cp = pltpu.make_async_copy(
            w_hbm.at[pl.ds(d * RD, RD), :],
            w_vmem.at[d // SPLIT, pl.ds((d % SPLIT) * RD, RD), :],
            in_sems.at[d])
        cp.start()
        copies.append(cp)
    src = src_ref[0]
    stores = []
    for q in range(NC):
        for s in range(SPLIT):
            copies[q * SPLIT + s].wait()
        y = jax.lax.dot_general(
            src, w_vmem[q],
            dimension_numbers=(((1,), (1,)), ((), ())),
            preferred_element_type=jnp.float32,
        )
        yb = y + b_ref[:, q * RQ:(q + 1) * RQ]
        out_vmem[q] = yb[None, :, :] - y[:, None, :]
        st = pltpu.make_async_copy(
            out_vmem.at[q],
            out_hbm.at[:, :, pl.ds(q * RQ, RQ)],
            out_sems.at[q])
        st.start()
        stores.append(st)
    for st in stores:
        st.wait()


def kernel(src, W, b):
    out = pl.pallas_call(
        _edge_enhance_kernel,
        in_specs=[
            pl.BlockSpec((1, JOINTS, EMB), lambda: (0, 0, 0)),
            pl.BlockSpec((1, EMB), lambda: (0, 0)),
            pl.BlockSpec(memory_space=pltpu.MemorySpace.HBM),
        ],
        out_specs=pl.BlockSpec(memory_space=pltpu.MemorySpace.HBM),
        out_shape=jax.ShapeDtypeStruct((JOINTS, JOINTS, EMB), jnp.float32),
        scratch_shapes=[
            pltpu.VMEM((NC, RQ, EMB), jnp.float32),
            pltpu.VMEM((NC, JOINTS, JOINTS, RQ), jnp.float32),
            pltpu.SemaphoreType.DMA((NDMA,)),
            pltpu.SemaphoreType.DMA((NC,)),
        ],
    )(src, b.reshape(1, EMB), W)
    return out
